# baseline (speedup 1.0000x reference)
import functools

import jax
import jax.numpy as jnp
from jax import lax
from jax.experimental import pallas as pl
from jax.experimental.pallas import tpu as pltpu
from jax.experimental.pallas import tpu_sc as plsc

VOCAB = 32768
HIDDEN = 256


def _make_sc_gather(B: int, D: int, n_chunks: int = 4):
    info = plsc.get_sparse_core_info()
    NC, NS = info.num_cores, info.num_subcores
    NW = NC * NS
    assert B % (8 * NW) == 0
    b_per_w = B // NW
    assert b_per_w % n_chunks == 0
    rows_c = b_per_w // n_chunks
    mesh = plsc.VectorSubcoreMesh(core_axis_name="c", subcore_axis_name="s")

    @functools.partial(
        pl.kernel,
        mesh=mesh,
        out_type=jax.ShapeDtypeStruct((B, D), jnp.float32),
        scratch_types=[
            pltpu.VMEM((b_per_w,), jnp.int32),
            pltpu.VMEM((n_chunks, rows_c, D), jnp.float32),
        ]
        + [pltpu.SemaphoreType.DMA] * (2 * n_chunks),
    )
    def gather_kernel(table_hbm, idx_hbm, out_hbm, idx_v, rows_v, *sems):
        gsem, wsem = sems[:n_chunks], sems[n_chunks:]
        wid = lax.axis_index("s") * NC + lax.axis_index("c")
        base = wid * b_per_w
        def gather(c):
            return pltpu.async_copy(
                table_hbm.at[idx_v.at[pl.ds(c * rows_c, rows_c)]],
                rows_v.at[c],
                gsem[c],
            )

        pltpu.sync_copy(idx_hbm.at[pl.ds(base, rows_c)], idx_v.at[pl.ds(0, rows_c)])
        gathers = [gather(0)]
        pltpu.sync_copy(
            idx_hbm.at[pl.ds(base + rows_c, b_per_w - rows_c)],
            idx_v.at[pl.ds(rows_c, b_per_w - rows_c)],
        )
        gathers += [gather(c) for c in range(1, n_chunks)]
        writes = []
        for c in range(n_chunks):
            gathers[c].wait()
            writes.append(
                pltpu.async_copy(
                    rows_v.at[c],
                    out_hbm.at[pl.ds(base + c * rows_c, rows_c)],
                    wsem[c],
                )
            )
        for w in writes:
            w.wait()

    return gather_kernel


def _mm_body(h_ref, w_ref, o_ref):
    h = h_ref[...].astype(jnp.bfloat16)
    w = w_ref[...].astype(jnp.bfloat16)
    o_ref[...] = lax.dot_general(
        h, w, (((1,), (1,)), ((), ())), preferred_element_type=jnp.float32
    )


def _projection(h, w, nt: int, vmem_limit_bytes: int | None = None):
    B, D = h.shape
    V = w.shape[0]
    grid = (pl.cdiv(V, nt),)
    params = {}
    if vmem_limit_bytes is not None:
        params["compiler_params"] = pltpu.CompilerParams(
            vmem_limit_bytes=vmem_limit_bytes
        )
    return pl.pallas_call(
        _mm_body,
        grid=grid,
        in_specs=[
            pl.BlockSpec((B, D), lambda i: (0, 0)),
            pl.BlockSpec((nt, D), lambda i: (i, 0)),
        ],
        out_specs=pl.BlockSpec((B, nt), lambda i: (0, i)),
        out_shape=jax.ShapeDtypeStruct((B, V), jnp.float32),
        **params,
    )(h, w)


def kernel(input_ids, embed, lm_head_w):
    Bt, S = input_ids.shape
    B = Bt * S
    ids = input_ids.reshape(B).astype(jnp.int32)
    h = _make_sc_gather(B, HIDDEN)(embed, ids)
    logits = _projection(h, lm_head_w, nt=1536, vmem_limit_bytes=100 * 1024 * 1024)
    return logits.reshape(Bt, S, VOCAB)
